# baseline (speedup 1.0000x reference)
import functools

import jax
import jax.numpy as jnp
from jax import lax
from jax.experimental import pallas as pl
from jax.experimental.pallas import tpu as pltpu
from jax.experimental.pallas import tpu_sc as plsc

N = 10000
E = 320000
ND = 128
ED = 16
HID = 128

_NC = 2
_NS = 16
_NW = _NC * _NS

_NCH = 4
_EC = E // _NCH
_GW = 128
_BE = 4000
_SK = 5
_ROWS = _EC // 128
_NCHUNK = _ROWS // _SK
_NITER = (_NCHUNK + _NW - 1) // _NW


def _vmesh():
    return plsc.VectorSubcoreMesh(core_axis_name="core", subcore_axis_name="subcore")


def _pre_body(x_ref, wa_ref, wb_ref, ta_ref, tb_ref):
    x = x_ref[...]
    ta_ref[...] = jnp.dot(x, wa_ref[...], preferred_element_type=jnp.float32)
    tb_ref[...] = jnp.dot(x, wb_ref[...], preferred_element_type=jnp.float32)


def _pre(x, wa, wb):
    return pl.pallas_call(
        _pre_body,
        out_shape=[jax.ShapeDtypeStruct((N, ND), jnp.float32)] * 2,
    )(x, wa, wb)


def _sc_gather(ta, tb, src, dst):
    @functools.partial(
        pl.kernel,
        out_type=[jax.ShapeDtypeStruct((_EC, ND), jnp.float32)] * 2,
        mesh=_vmesh(),
    )
    def k(ta_hbm, tb_hbm, src_hbm, dst_hbm, a_hbm, b_hbm):
        def body(s_vmem, d_vmem, a_vmem, b_vmem):
            pltpu.sync_copy(ta_hbm.at[s_vmem.at[0]], a_vmem)
            pltpu.sync_copy(tb_hbm.at[d_vmem.at[0]], b_vmem)

        pltpu.emit_pipeline(
            body,
            grid=(_EC // _GW,),
            in_specs=[
                pl.BlockSpec((1, _GW), lambda i: (0, i)),
                pl.BlockSpec((1, _GW), lambda i: (0, i)),
            ],
            out_specs=[
                pl.BlockSpec((_GW, ND), lambda i: (i, 0)),
                pl.BlockSpec((_GW, ND), lambda i: (i, 0)),
            ],
            core_axis_name=("core", "subcore"),
            dimension_semantics=(pltpu.PARALLEL,),
        )(src_hbm, dst_hbm, a_hbm, b_hbm)

    return k(ta, tb, src, dst)


def _edge_mlp_body(a_ref, b_ref, ea_ref, wc_ref, b1_ref, g_ref, bt_ref,
                   w2_ref, b2_ref, o_ref):
    ea = ea_ref[...]
    h = (a_ref[...] + b_ref[...]
         + jnp.dot(ea, wc_ref[...], preferred_element_type=jnp.float32)
         + b1_ref[...])
    mu = jnp.mean(h, axis=-1, keepdims=True)
    hc = h - mu
    var = jnp.mean(hc * hc, axis=-1, keepdims=True)
    hn = hc * lax.rsqrt(var + 1e-5) * g_ref[...] + bt_ref[...]
    hs = hn / (1.0 + jnp.exp(-hn))
    o_ref[...] = (jnp.dot(hs, w2_ref[...], preferred_element_type=jnp.float32)
                  + b2_ref[...] + ea)


def _edge_mlp(a, b, ea, wc, b1, g, bt, w2, b2):
    return pl.pallas_call(
        _edge_mlp_body,
        grid=(_EC // _BE,),
        in_specs=[
            pl.BlockSpec((_BE, ND), lambda i: (i, 0)),
            pl.BlockSpec((_BE, ND), lambda i: (i, 0)),
            pl.BlockSpec((_BE, ED), lambda i: (i, 0)),
            pl.BlockSpec((ED, HID), lambda i: (0, 0)),
            pl.BlockSpec((1, HID), lambda i: (0, 0)),
            pl.BlockSpec((1, HID), lambda i: (0, 0)),
            pl.BlockSpec((1, HID), lambda i: (0, 0)),
            pl.BlockSpec((HID, ED), lambda i: (0, 0)),
            pl.BlockSpec((1, ED), lambda i: (0, 0)),
        ],
        out_specs=pl.BlockSpec((_BE, ED), lambda i: (i, 0)),
        out_shape=jax.ShapeDtypeStruct((_EC, ED), jnp.float32),
    )(a, b, ea, wc, b1, g, bt, w2, b2)


def _sc_scatter(edge_new, dst2, zeros):
    @functools.partial(
        pl.kernel,
        out_type=jax.ShapeDtypeStruct((_NC, N, ED), jnp.float32),
        mesh=_vmesh(),
        compiler_params=pltpu.CompilerParams(use_tc_tiling_on_sc=False),
        scratch_types=[
            pltpu.VMEM_SHARED((N, ED), jnp.float32),
            pltpu.VMEM((_SK, 128), jnp.int32),
            pltpu.VMEM((_SK * 128, ED), jnp.float32),
        ],
    )
    def k(en_hbm, d2_hbm, z_hbm, p_hbm, shared, idx_v, data_v):
        cid = lax.axis_index("core")
        sid = lax.axis_index("subcore")
        wid = sid * _NC + cid

        @pl.when(sid == 0)
        def _():
            pltpu.sync_copy(z_hbm, shared)

        plsc.subcore_barrier()

        @pl.loop(0, _NITER)
        def _(t):
            c = wid + t * _NW

            @pl.when(c < _NCHUNK)
            def _():
                pltpu.sync_copy(d2_hbm.at[pl.ds(c * _SK, _SK)], idx_v)
                pltpu.sync_copy(en_hbm.at[pl.ds(c * _SK * 128, _SK * 128)], data_v)
                for j in range(_SK):
                    pltpu.sync_copy(data_v.at[pl.ds(j * 128, 128)],
                                    shared.at[idx_v.at[j]], add=True)

        plsc.subcore_barrier()

        @pl.when(sid == 0)
        def _():
            pltpu.sync_copy(shared, p_hbm.at[cid])

    return k(edge_new, dst2, zeros)


def _node_mlp_body(x_ref, p_ref, w1a_ref, w1b_ref, b1_ref, g_ref, bt_ref,
                   w2_ref, b2_ref, o_ref):
    x = x_ref[...]
    agg = p_ref[0]
    for i in range(1, 2 * _NCH):
        agg = agg + p_ref[i]
    h = (jnp.dot(x, w1a_ref[...], preferred_element_type=jnp.float32)
         + jnp.dot(agg, w1b_ref[...], preferred_element_type=jnp.float32)
         + b1_ref[...])
    mu = jnp.mean(h, axis=-1, keepdims=True)
    hc = h - mu
    var = jnp.mean(hc * hc, axis=-1, keepdims=True)
    hn = hc * lax.rsqrt(var + 1e-5) * g_ref[...] + bt_ref[...]
    hs = hn / (1.0 + jnp.exp(-hn))
    o_ref[...] = (jnp.dot(hs, w2_ref[...], preferred_element_type=jnp.float32)
                  + b2_ref[...] + x)


def _node_mlp(x, parts, w1a, w1b, b1, g, bt, w2, b2):
    return pl.pallas_call(
        _node_mlp_body,
        out_shape=jax.ShapeDtypeStruct((N, ND), jnp.float32),
    )(x, parts, w1a, w1b, b1, g, bt, w2, b2)


def kernel(x, edge_index, edge_attr, eW1, eb1, eg, ebt, eW2, eb2,
           nW1, nb1, ng, nbt, nW2, nb2):
    src = edge_index[0]
    dst = edge_index[1]
    wa = eW1[:ND]
    wb = eW1[ND:2 * ND]
    wc = eW1[2 * ND:]
    eb1r = eb1.reshape(1, HID)
    egr = eg.reshape(1, HID)
    ebtr = ebt.reshape(1, HID)
    eb2r = eb2.reshape(1, ED)

    ta, tb = _pre(x, wa, wb)
    zeros = jnp.zeros((N, ED), jnp.float32)

    edge_outs = []
    parts = []
    for c in range(_NCH):
        lo = c * _EC
        s_c = lax.dynamic_slice(src, (lo,), (_EC,)).reshape(1, _EC)
        d_c = lax.dynamic_slice(dst, (lo,), (_EC,)).reshape(1, _EC)
        ea_c = lax.dynamic_slice(edge_attr, (lo, 0), (_EC, ED))
        ga, gb = _sc_gather(ta, tb, s_c, d_c)
        en_c = _edge_mlp(ga, gb, ea_c, wc, eb1r, egr, ebtr, eW2, eb2r)
        d2_c = lax.dynamic_slice(dst, (lo,), (_EC,)).reshape(_EC // 128, 128)
        parts.append(_sc_scatter(en_c, d2_c, zeros))
        edge_outs.append(en_c)

    edge_new = jnp.concatenate(edge_outs, axis=0)
    pstack = jnp.concatenate(parts, axis=0)

    x_new = _node_mlp(x, pstack, nW1[:ND], nW1[ND:],
                      nb1.reshape(1, HID), ng.reshape(1, HID),
                      nbt.reshape(1, HID), nW2, nb2.reshape(1, ND))
    return x_new, edge_new


# speedup vs baseline: 1.0612x; 1.0612x over previous
import functools

import jax
import jax.numpy as jnp
from jax import lax
from jax.experimental import pallas as pl
from jax.experimental.pallas import tpu as pltpu
from jax.experimental.pallas import tpu_sc as plsc

N = 10000
E = 320000
ND = 128
ED = 16
HID = 128

_NC = 2
_NS = 16
_NW = _NC * _NS

_NCH = 4
_EC = E // _NCH
_GW = 128
_BE = 3200
_PB = _BE // 8
_SKR = _BE // 128
_ROWS = _EC // 128
_NCHUNK = _EC // _BE
_NITER = (_NCHUNK + _NW - 1) // _NW


def _vmesh():
    return plsc.VectorSubcoreMesh(core_axis_name="core", subcore_axis_name="subcore")


def _pre_body(x_ref, wa_ref, wb_ref, ta_ref, tb_ref):
    x = x_ref[...]
    ta_ref[...] = jnp.dot(x, wa_ref[...], preferred_element_type=jnp.float32)
    tb_ref[...] = jnp.dot(x, wb_ref[...], preferred_element_type=jnp.float32)


def _pre(x, wa, wb):
    return pl.pallas_call(
        _pre_body,
        out_shape=[jax.ShapeDtypeStruct((N, ND), jnp.float32)] * 2,
    )(x, wa, wb)


def _sc_gather(ta, tb, src, dst, c):
    base = c * (_EC // _GW)

    @functools.partial(
        pl.kernel,
        out_type=[jax.ShapeDtypeStruct((_EC, ND), jnp.float32)] * 2,
        mesh=_vmesh(),
    )
    def k(ta_hbm, tb_hbm, src_hbm, dst_hbm, a_hbm, b_hbm):
        def body(s_vmem, d_vmem, a_vmem, b_vmem):
            pltpu.sync_copy(ta_hbm.at[s_vmem.at[0]], a_vmem)
            pltpu.sync_copy(tb_hbm.at[d_vmem.at[0]], b_vmem)

        pltpu.emit_pipeline(
            body,
            grid=(_EC // _GW,),
            in_specs=[
                pl.BlockSpec((1, _GW), lambda i: (0, base + i)),
                pl.BlockSpec((1, _GW), lambda i: (0, base + i)),
            ],
            out_specs=[
                pl.BlockSpec((_GW, ND), lambda i: (i, 0)),
                pl.BlockSpec((_GW, ND), lambda i: (i, 0)),
            ],
            core_axis_name=("core", "subcore"),
            dimension_semantics=(pltpu.PARALLEL,),
        )(src_hbm, dst_hbm, a_hbm, b_hbm)

    return k(ta, tb, src, dst)


def _edge_mlp_body(a_ref, b_ref, ea_ref, wc_ref, b1_ref, g_ref, bt_ref,
                   w2_ref, b2_ref, acc_ref, o_ref, op_ref):
    del acc_ref
    ea = ea_ref[...]
    h = (a_ref[...] + b_ref[...]
         + jnp.dot(ea, wc_ref[...], preferred_element_type=jnp.float32)
         + b1_ref[...])
    mu = jnp.mean(h, axis=-1, keepdims=True)
    hc = h - mu
    var = jnp.mean(hc * hc, axis=-1, keepdims=True)
    hn = hc * lax.rsqrt(var + 1e-5) * g_ref[...] + bt_ref[...]
    hs = hn / (1.0 + jnp.exp(-hn))
    y = (jnp.dot(hs, w2_ref[...], preferred_element_type=jnp.float32)
         + b2_ref[...] + ea)
    o_ref[...] = y
    op_ref[...] = jnp.concatenate(
        [y[m * _PB:(m + 1) * _PB, :] for m in range(8)], axis=1)


def _edge_mlp(a, b, ea, wc, b1, g, bt, w2, b2, acc, c):
    base = c * (_EC // _BE)
    return pl.pallas_call(
        _edge_mlp_body,
        grid=(_EC // _BE,),
        in_specs=[
            pl.BlockSpec((_BE, ND), lambda i: (i, 0)),
            pl.BlockSpec((_BE, ND), lambda i: (i, 0)),
            pl.BlockSpec((_BE, ED), lambda i: (base + i, 0)),
            pl.BlockSpec((ED, HID), lambda i: (0, 0)),
            pl.BlockSpec((1, HID), lambda i: (0, 0)),
            pl.BlockSpec((1, HID), lambda i: (0, 0)),
            pl.BlockSpec((1, HID), lambda i: (0, 0)),
            pl.BlockSpec((HID, ED), lambda i: (0, 0)),
            pl.BlockSpec((1, ED), lambda i: (0, 0)),
            pl.BlockSpec(memory_space=pl.ANY),
        ],
        out_specs=[pl.BlockSpec((_BE, ED), lambda i: (base + i, 0)),
                   pl.BlockSpec((_PB, 128), lambda i: (i, 0))],
        out_shape=[jax.ShapeDtypeStruct((E, ED), jnp.float32),
                   jax.ShapeDtypeStruct((_EC // 8, 128), jnp.float32)],
        input_output_aliases={9: 0},
    )(a, b, ea, wc, b1, g, bt, w2, b2, acc)


def _sc_scatter(en_pack, dst2, zeros, c):
    rbase = c * _ROWS

    @functools.partial(
        pl.kernel,
        out_type=jax.ShapeDtypeStruct((_NC, N, ED), jnp.float32),
        mesh=_vmesh(),
        compiler_params=pltpu.CompilerParams(use_tc_tiling_on_sc=False),
        scratch_types=[
            pltpu.VMEM_SHARED((N, ED), jnp.float32),
            pltpu.VMEM((_SKR, 128), jnp.int32),
            pltpu.VMEM((_PB, 128), jnp.float32),
            pltpu.VMEM((_BE, ED), jnp.float32),
        ],
    )
    def k(ep_hbm, d2_hbm, z_hbm, p_hbm, shared, idx_v, pack_v, data_v):
        cid = lax.axis_index("core")
        sid = lax.axis_index("subcore")
        wid = sid * _NC + cid

        @pl.when(sid == 0)
        def _():
            pltpu.sync_copy(z_hbm, shared)

        plsc.subcore_barrier()

        @pl.when(wid < _NCHUNK)
        def _():
            cc = wid
            pltpu.sync_copy(d2_hbm.at[pl.ds(rbase + cc * _SKR, _SKR)], idx_v)
            pltpu.sync_copy(ep_hbm.at[pl.ds(cc * _PB, _PB)], pack_v)

            @pl.loop(0, _PB)
            def _(r):
                for m in range(8):
                    data_v[m * _PB + r, :] = pack_v[r, pl.ds(m * 16, 16)]

            for j in range(_SKR):
                pltpu.sync_copy(data_v.at[pl.ds(j * 128, 128)],
                                shared.at[idx_v.at[j]], add=True)

        plsc.subcore_barrier()

        @pl.when(sid == 0)
        def _():
            pltpu.sync_copy(shared, p_hbm.at[cid])

    return k(en_pack, dst2, zeros)


def _node_mlp_body(x_ref, p_ref, w1a_ref, w1b_ref, b1_ref, g_ref, bt_ref,
                   w2_ref, b2_ref, o_ref):
    x = x_ref[...]
    agg = p_ref[0]
    for i in range(1, 2 * _NCH):
        agg = agg + p_ref[i]
    h = (jnp.dot(x, w1a_ref[...], preferred_element_type=jnp.float32)
         + jnp.dot(agg, w1b_ref[...], preferred_element_type=jnp.float32)
         + b1_ref[...])
    mu = jnp.mean(h, axis=-1, keepdims=True)
    hc = h - mu
    var = jnp.mean(hc * hc, axis=-1, keepdims=True)
    hn = hc * lax.rsqrt(var + 1e-5) * g_ref[...] + bt_ref[...]
    hs = hn / (1.0 + jnp.exp(-hn))
    o_ref[...] = (jnp.dot(hs, w2_ref[...], preferred_element_type=jnp.float32)
                  + b2_ref[...] + x)


def _node_mlp(x, parts, w1a, w1b, b1, g, bt, w2, b2):
    return pl.pallas_call(
        _node_mlp_body,
        out_shape=jax.ShapeDtypeStruct((N, ND), jnp.float32),
    )(x, parts, w1a, w1b, b1, g, bt, w2, b2)


def kernel(x, edge_index, edge_attr, eW1, eb1, eg, ebt, eW2, eb2,
           nW1, nb1, ng, nbt, nW2, nb2):
    src = edge_index[0].reshape(1, E)
    dst = edge_index[1].reshape(1, E)
    dst2 = edge_index[1].reshape(E // 128, 128)
    wa = eW1[:ND]
    wb = eW1[ND:2 * ND]
    wc = eW1[2 * ND:]
    eb1r = eb1.reshape(1, HID)
    egr = eg.reshape(1, HID)
    ebtr = ebt.reshape(1, HID)
    eb2r = eb2.reshape(1, ED)

    ta, tb = _pre(x, wa, wb)
    zeros = jnp.zeros((N, ED), jnp.float32)

    acc = jnp.zeros((E, ED), jnp.float32)
    parts = []
    for c in range(_NCH):
        ga, gb = _sc_gather(ta, tb, src, dst, c)
        acc, ep_c = _edge_mlp(ga, gb, edge_attr, wc, eb1r, egr, ebtr,
                              eW2, eb2r, acc, c)
        parts.append(_sc_scatter(ep_c, dst2, zeros, c))

    edge_new = acc
    pstack = jnp.concatenate(parts, axis=0)

    x_new = _node_mlp(x, pstack, nW1[:ND], nW1[ND:],
                      nb1.reshape(1, HID), ng.reshape(1, HID),
                      nbt.reshape(1, HID), nW2, nb2.reshape(1, ND))
    return x_new, edge_new


# speedup vs baseline: 1.0895x; 1.0267x over previous
import functools

import jax
import jax.numpy as jnp
from jax import lax
from jax.experimental import pallas as pl
from jax.experimental.pallas import tpu as pltpu
from jax.experimental.pallas import tpu_sc as plsc

N = 10000
E = 320000
ND = 128
ED = 16
HID = 128

_NC = 2
_NS = 16
_NW = _NC * _NS

_NCH = 4
_EC = E // _NCH
_GW = 128
_BE = 3200
_PB = _BE // 8
_SKR = _BE // 128
_ROWS = _EC // 128
_NCHUNK = _EC // _BE
_NITER = (_NCHUNK + _NW - 1) // _NW


def _vmesh():
    return plsc.VectorSubcoreMesh(core_axis_name="core", subcore_axis_name="subcore")


def _pre_body(x_ref, wa_ref, wb_ref, ta_ref, tb_ref):
    x = x_ref[...]
    ta_ref[...] = jnp.dot(x, wa_ref[...], preferred_element_type=jnp.float32)
    tb_ref[...] = jnp.dot(x, wb_ref[...], preferred_element_type=jnp.float32)


def _pre(x, wa, wb):
    return pl.pallas_call(
        _pre_body,
        out_shape=[jax.ShapeDtypeStruct((N, ND), jnp.float32)] * 2,
    )(x, wa, wb)


def _sc_gather(ta, tb, src, dst, c):
    base = c * (_EC // _GW)

    @functools.partial(
        pl.kernel,
        out_type=[jax.ShapeDtypeStruct((_EC, ND), jnp.float32)] * 2,
        mesh=_vmesh(),
    )
    def k(ta_hbm, tb_hbm, src_hbm, dst_hbm, a_hbm, b_hbm):
        def body(s_vmem, d_vmem, a_vmem, b_vmem):
            pltpu.sync_copy(ta_hbm.at[s_vmem.at[0]], a_vmem)
            pltpu.sync_copy(tb_hbm.at[d_vmem.at[0]], b_vmem)

        pltpu.emit_pipeline(
            body,
            grid=(_EC // _GW,),
            in_specs=[
                pl.BlockSpec((1, _GW), lambda i: (0, base + i)),
                pl.BlockSpec((1, _GW), lambda i: (0, base + i)),
            ],
            out_specs=[
                pl.BlockSpec((_GW, ND), lambda i: (i, 0)),
                pl.BlockSpec((_GW, ND), lambda i: (i, 0)),
            ],
            core_axis_name=("core", "subcore"),
            dimension_semantics=(pltpu.PARALLEL,),
        )(src_hbm, dst_hbm, a_hbm, b_hbm)

    return k(ta, tb, src, dst)


def _edge_mlp_body(a_ref, b_ref, ea_ref, wc_ref, b1_ref, g_ref, bt_ref,
                   w2_ref, b2_ref, op_ref):
    ea = ea_ref[...]
    h = (a_ref[...] + b_ref[...]
         + jnp.dot(ea, wc_ref[...], preferred_element_type=jnp.float32)
         + b1_ref[...])
    mu = jnp.mean(h, axis=-1, keepdims=True)
    hc = h - mu
    var = jnp.mean(hc * hc, axis=-1, keepdims=True)
    hn = hc * lax.rsqrt(var + 1e-5) * g_ref[...] + bt_ref[...]
    hs = hn / (1.0 + jnp.exp(-hn))
    y = (jnp.dot(hs, w2_ref[...], preferred_element_type=jnp.float32)
         + b2_ref[...] + ea)
    op_ref[...] = jnp.concatenate(
        [y[m * _PB:(m + 1) * _PB, :] for m in range(8)], axis=1)


def _edge_mlp(a, b, ea, wc, b1, g, bt, w2, b2, c):
    base = c * (_EC // _BE)
    return pl.pallas_call(
        _edge_mlp_body,
        grid=(_EC // _BE,),
        in_specs=[
            pl.BlockSpec((_BE, ND), lambda i: (i, 0)),
            pl.BlockSpec((_BE, ND), lambda i: (i, 0)),
            pl.BlockSpec((_BE, ED), lambda i: (base + i, 0)),
            pl.BlockSpec((ED, HID), lambda i: (0, 0)),
            pl.BlockSpec((1, HID), lambda i: (0, 0)),
            pl.BlockSpec((1, HID), lambda i: (0, 0)),
            pl.BlockSpec((1, HID), lambda i: (0, 0)),
            pl.BlockSpec((HID, ED), lambda i: (0, 0)),
            pl.BlockSpec((1, ED), lambda i: (0, 0)),
        ],
        out_specs=pl.BlockSpec((_PB, 128), lambda i: (i, 0)),
        out_shape=jax.ShapeDtypeStruct((_EC // 8, 128), jnp.float32),
    )(a, b, ea, wc, b1, g, bt, w2, b2)


def _unpack_body(p_ref, o_ref):
    p = p_ref[...]
    o_ref[...] = jnp.concatenate(
        [p[:, m * 16:(m + 1) * 16] for m in range(8)], axis=0)


def _unpack(ep_all):
    return pl.pallas_call(
        _unpack_body,
        grid=(E // _BE,),
        in_specs=[pl.BlockSpec((_PB, 128), lambda i: (i, 0))],
        out_specs=pl.BlockSpec((_BE, ED), lambda i: (i, 0)),
        out_shape=jax.ShapeDtypeStruct((E, ED), jnp.float32),
    )(ep_all)


def _sc_scatter(en_pack, dst2, zeros, c):
    rbase = c * _ROWS

    @functools.partial(
        pl.kernel,
        out_type=jax.ShapeDtypeStruct((_NC, N, ED), jnp.float32),
        mesh=_vmesh(),
        compiler_params=pltpu.CompilerParams(use_tc_tiling_on_sc=False),
        scratch_types=[
            pltpu.VMEM_SHARED((N, ED), jnp.float32),
            pltpu.VMEM((_SKR, 128), jnp.int32),
            pltpu.VMEM((_PB, 128), jnp.float32),
            pltpu.VMEM((_BE, ED), jnp.float32),
        ],
    )
    def k(ep_hbm, d2_hbm, z_hbm, p_hbm, shared, idx_v, pack_v, data_v):
        cid = lax.axis_index("core")
        sid = lax.axis_index("subcore")
        wid = sid * _NC + cid

        @pl.when(sid == 0)
        def _():
            pltpu.sync_copy(z_hbm, shared)

        plsc.subcore_barrier()

        @pl.when(wid < _NCHUNK)
        def _():
            cc = wid
            pltpu.sync_copy(d2_hbm.at[pl.ds(rbase + cc * _SKR, _SKR)], idx_v)
            pltpu.sync_copy(ep_hbm.at[pl.ds(cc * _PB, _PB)], pack_v)

            @pl.loop(0, _PB)
            def _(r):
                for m in range(8):
                    data_v[m * _PB + r, :] = pack_v[r, pl.ds(m * 16, 16)]

            for j in range(_SKR):
                pltpu.sync_copy(data_v.at[pl.ds(j * 128, 128)],
                                shared.at[idx_v.at[j]], add=True)

        plsc.subcore_barrier()

        @pl.when(sid == 0)
        def _():
            pltpu.sync_copy(shared, p_hbm.at[cid])

    return k(en_pack, dst2, zeros)


def _node_mlp_body(x_ref, p0_ref, p1_ref, p2_ref, p3_ref, w1a_ref, w1b_ref,
                   b1_ref, g_ref, bt_ref, w2_ref, b2_ref, o_ref):
    x = x_ref[...]
    agg = p0_ref[0] + p0_ref[1]
    for pr in (p1_ref, p2_ref, p3_ref):
        agg = agg + pr[0] + pr[1]
    h = (jnp.dot(x, w1a_ref[...], preferred_element_type=jnp.float32)
         + jnp.dot(agg, w1b_ref[...], preferred_element_type=jnp.float32)
         + b1_ref[...])
    mu = jnp.mean(h, axis=-1, keepdims=True)
    hc = h - mu
    var = jnp.mean(hc * hc, axis=-1, keepdims=True)
    hn = hc * lax.rsqrt(var + 1e-5) * g_ref[...] + bt_ref[...]
    hs = hn / (1.0 + jnp.exp(-hn))
    o_ref[...] = (jnp.dot(hs, w2_ref[...], preferred_element_type=jnp.float32)
                  + b2_ref[...] + x)


def _node_mlp(x, parts, w1a, w1b, b1, g, bt, w2, b2):
    return pl.pallas_call(
        _node_mlp_body,
        out_shape=jax.ShapeDtypeStruct((N, ND), jnp.float32),
    )(x, *parts, w1a, w1b, b1, g, bt, w2, b2)


def kernel(x, edge_index, edge_attr, eW1, eb1, eg, ebt, eW2, eb2,
           nW1, nb1, ng, nbt, nW2, nb2):
    src = edge_index[0].reshape(1, E)
    dst = edge_index[1].reshape(1, E)
    dst2 = edge_index[1].reshape(E // 128, 128)
    wa = eW1[:ND]
    wb = eW1[ND:2 * ND]
    wc = eW1[2 * ND:]
    eb1r = eb1.reshape(1, HID)
    egr = eg.reshape(1, HID)
    ebtr = ebt.reshape(1, HID)
    eb2r = eb2.reshape(1, ED)

    ta, tb = _pre(x, wa, wb)
    zeros = jnp.zeros((N, ED), jnp.float32)

    parts = []
    eps = []
    for c in range(_NCH):
        ga, gb = _sc_gather(ta, tb, src, dst, c)
        ep_c = _edge_mlp(ga, gb, edge_attr, wc, eb1r, egr, ebtr,
                         eW2, eb2r, c)
        parts.append(_sc_scatter(ep_c, dst2, zeros, c))
        eps.append(ep_c)

    edge_new = _unpack(jnp.concatenate(eps, axis=0))

    x_new = _node_mlp(x, parts, nW1[:ND], nW1[ND:],
                      nb1.reshape(1, HID), ng.reshape(1, HID),
                      nbt.reshape(1, HID), nW2, nb2.reshape(1, ND))
    return x_new, edge_new


# speedup vs baseline: 1.0940x; 1.0041x over previous
import functools

import jax
import jax.numpy as jnp
from jax import lax
from jax.experimental import pallas as pl
from jax.experimental.pallas import tpu as pltpu
from jax.experimental.pallas import tpu_sc as plsc

N = 10000
E = 320000
ND = 128
ED = 16
HID = 128

_NC = 2
_NS = 16
_NW = _NC * _NS

_NCH = 4
_EC = E // _NCH
_GW = 128
_BE = 3200
_PB = _BE // 8
_SKR = _BE // 128
_ROWS = _EC // 128
_NCHUNK = _EC // _BE
_NITER = (_NCHUNK + _NW - 1) // _NW


def _vmesh():
    return plsc.VectorSubcoreMesh(core_axis_name="core", subcore_axis_name="subcore")


def _pre_body(x_ref, wa_ref, wb_ref, ta_ref, tb_ref):
    x = x_ref[...]
    ta_ref[...] = jnp.dot(x, wa_ref[...], preferred_element_type=jnp.float32)
    tb_ref[...] = jnp.dot(x, wb_ref[...], preferred_element_type=jnp.float32)


def _pre(x, wa, wb):
    return pl.pallas_call(
        _pre_body,
        out_shape=[jax.ShapeDtypeStruct((N, ND), jnp.float32)] * 2,
    )(x, wa, wb)


def _sc_gather(ta, tb, src, dst, c):
    base = c * (_EC // _GW)

    @functools.partial(
        pl.kernel,
        out_type=[jax.ShapeDtypeStruct((_EC, ND), jnp.float32)] * 2,
        mesh=_vmesh(),
    )
    def k(ta_hbm, tb_hbm, src_hbm, dst_hbm, a_hbm, b_hbm):
        def body(s_vmem, d_vmem, a_vmem, b_vmem):
            pltpu.sync_copy(ta_hbm.at[s_vmem.at[0]], a_vmem)
            pltpu.sync_copy(tb_hbm.at[d_vmem.at[0]], b_vmem)

        pltpu.emit_pipeline(
            body,
            grid=(_EC // _GW,),
            in_specs=[
                pl.BlockSpec((1, _GW), lambda i: (base + i, 0)),
                pl.BlockSpec((1, _GW), lambda i: (base + i, 0)),
            ],
            out_specs=[
                pl.BlockSpec((_GW, ND), lambda i: (i, 0)),
                pl.BlockSpec((_GW, ND), lambda i: (i, 0)),
            ],
            core_axis_name=("core", "subcore"),
            dimension_semantics=(pltpu.PARALLEL,),
        )(src_hbm, dst_hbm, a_hbm, b_hbm)

    return k(ta, tb, src, dst)


def _edge_mlp_body(a_ref, b_ref, ea_ref, wc_ref, b1_ref, g_ref, bt_ref,
                   w2_ref, b2_ref, op_ref):
    ea = ea_ref[...]
    h = (a_ref[...] + b_ref[...]
         + jnp.dot(ea, wc_ref[...], preferred_element_type=jnp.float32)
         + b1_ref[...])
    mu = jnp.mean(h, axis=-1, keepdims=True)
    hc = h - mu
    var = jnp.mean(hc * hc, axis=-1, keepdims=True)
    hn = hc * lax.rsqrt(var + 1e-5) * g_ref[...] + bt_ref[...]
    hs = hn / (1.0 + jnp.exp(-hn))
    y = (jnp.dot(hs, w2_ref[...], preferred_element_type=jnp.float32)
         + b2_ref[...] + ea)
    op_ref[...] = jnp.concatenate(
        [y[m * _PB:(m + 1) * _PB, :] for m in range(8)], axis=1)


def _edge_mlp(a, b, ea, wc, b1, g, bt, w2, b2, c):
    base = c * (_EC // _BE)
    return pl.pallas_call(
        _edge_mlp_body,
        grid=(_EC // _BE,),
        in_specs=[
            pl.BlockSpec((_BE, ND), lambda i: (i, 0)),
            pl.BlockSpec((_BE, ND), lambda i: (i, 0)),
            pl.BlockSpec((_BE, ED), lambda i: (base + i, 0)),
            pl.BlockSpec((ED, HID), lambda i: (0, 0)),
            pl.BlockSpec((1, HID), lambda i: (0, 0)),
            pl.BlockSpec((1, HID), lambda i: (0, 0)),
            pl.BlockSpec((1, HID), lambda i: (0, 0)),
            pl.BlockSpec((HID, ED), lambda i: (0, 0)),
            pl.BlockSpec((1, ED), lambda i: (0, 0)),
        ],
        out_specs=pl.BlockSpec((_PB, 128), lambda i: (i, 0)),
        out_shape=jax.ShapeDtypeStruct((_EC // 8, 128), jnp.float32),
    )(a, b, ea, wc, b1, g, bt, w2, b2)


def _unpack_body(p_ref, o_ref):
    p = p_ref[...]
    o_ref[...] = jnp.concatenate(
        [p[:, m * 16:(m + 1) * 16] for m in range(8)], axis=0)


def _unpack(ep_all):
    return pl.pallas_call(
        _unpack_body,
        grid=(E // _BE,),
        in_specs=[pl.BlockSpec((_PB, 128), lambda i: (i, 0))],
        out_specs=pl.BlockSpec((_BE, ED), lambda i: (i, 0)),
        out_shape=jax.ShapeDtypeStruct((E, ED), jnp.float32),
    )(ep_all)


def _sc_scatter(en_pack, dst2, zeros, c):
    rbase = c * _ROWS

    @functools.partial(
        pl.kernel,
        out_type=jax.ShapeDtypeStruct((_NC, N, ED), jnp.float32),
        mesh=_vmesh(),
        compiler_params=pltpu.CompilerParams(use_tc_tiling_on_sc=False),
        scratch_types=[
            pltpu.VMEM_SHARED((N, ED), jnp.float32),
            pltpu.VMEM((_SKR, 128), jnp.int32),
            pltpu.VMEM((_PB, 128), jnp.float32),
            pltpu.VMEM((_BE, ED), jnp.float32),
        ],
    )
    def k(ep_hbm, d2_hbm, z_hbm, p_hbm, shared, idx_v, pack_v, data_v):
        cid = lax.axis_index("core")
        sid = lax.axis_index("subcore")
        wid = sid * _NC + cid

        @pl.when(sid == 0)
        def _():
            pltpu.sync_copy(z_hbm, shared)

        plsc.subcore_barrier()

        @pl.when(wid < _NCHUNK)
        def _():
            cc = wid
            pltpu.sync_copy(d2_hbm.at[pl.ds(rbase + cc * _SKR, _SKR)], idx_v)
            pltpu.sync_copy(ep_hbm.at[pl.ds(cc * _PB, _PB)], pack_v)

            @pl.loop(0, _PB)
            def _(r):
                for m in range(8):
                    data_v[m * _PB + r, :] = pack_v[r, pl.ds(m * 16, 16)]

            for j in range(_SKR):
                pltpu.sync_copy(data_v.at[pl.ds(j * 128, 128)],
                                shared.at[idx_v.at[j]], add=True)

        plsc.subcore_barrier()

        @pl.when(sid == 0)
        def _():
            pltpu.sync_copy(shared, p_hbm.at[cid])

    return k(en_pack, dst2, zeros)


def _node_mlp_body(x_ref, p0_ref, p1_ref, p2_ref, p3_ref, w1a_ref, w1b_ref,
                   b1_ref, g_ref, bt_ref, w2_ref, b2_ref, o_ref):
    x = x_ref[...]
    agg = p0_ref[0] + p0_ref[1]
    for pr in (p1_ref, p2_ref, p3_ref):
        agg = agg + pr[0] + pr[1]
    h = (jnp.dot(x, w1a_ref[...], preferred_element_type=jnp.float32)
         + jnp.dot(agg, w1b_ref[...], preferred_element_type=jnp.float32)
         + b1_ref[...])
    mu = jnp.mean(h, axis=-1, keepdims=True)
    hc = h - mu
    var = jnp.mean(hc * hc, axis=-1, keepdims=True)
    hn = hc * lax.rsqrt(var + 1e-5) * g_ref[...] + bt_ref[...]
    hs = hn / (1.0 + jnp.exp(-hn))
    o_ref[...] = (jnp.dot(hs, w2_ref[...], preferred_element_type=jnp.float32)
                  + b2_ref[...] + x)


def _node_mlp(x, parts, w1a, w1b, b1, g, bt, w2, b2):
    return pl.pallas_call(
        _node_mlp_body,
        out_shape=jax.ShapeDtypeStruct((N, ND), jnp.float32),
    )(x, *parts, w1a, w1b, b1, g, bt, w2, b2)


def kernel(x, edge_index, edge_attr, eW1, eb1, eg, ebt, eW2, eb2,
           nW1, nb1, ng, nbt, nW2, nb2):
    src = edge_index[0].reshape(E // 128, 128)
    dst2 = edge_index[1].reshape(E // 128, 128)
    dst = dst2
    wa = eW1[:ND]
    wb = eW1[ND:2 * ND]
    wc = eW1[2 * ND:]
    eb1r = eb1.reshape(1, HID)
    egr = eg.reshape(1, HID)
    ebtr = ebt.reshape(1, HID)
    eb2r = eb2.reshape(1, ED)

    ta, tb = _pre(x, wa, wb)
    zeros = jnp.zeros((N, ED), jnp.float32)

    parts = []
    eps = []
    for c in range(_NCH):
        ga, gb = _sc_gather(ta, tb, src, dst, c)
        ep_c = _edge_mlp(ga, gb, edge_attr, wc, eb1r, egr, ebtr,
                         eW2, eb2r, c)
        parts.append(_sc_scatter(ep_c, dst2, zeros, c))
        eps.append(ep_c)

    edge_new = _unpack(jnp.concatenate(eps, axis=0))

    x_new = _node_mlp(x, parts, nW1[:ND], nW1[ND:],
                      nb1.reshape(1, HID), ng.reshape(1, HID),
                      nbt.reshape(1, HID), nW2, nb2.reshape(1, ND))
    return x_new, edge_new
